# baseline (speedup 1.0000x reference)
import jax
import jax.numpy as jnp
from jax.experimental import pallas as pl
from jax.experimental.pallas import tpu as pltpu

_EPS = 1e-6


def _mod_kernel(cond_ref, w_ada_ref, b_ada_ref, mod_ref):
    c = cond_ref[...]
    s = c * jax.nn.sigmoid(c)
    mod_ref[...] = (
        jnp.dot(s, w_ada_ref[...], preferred_element_type=jnp.float32)
        + b_ada_ref[...]
    )


def _main_body(x_ref, mod_ref, w_proj_ref, b_proj_ref, out_ref):
    d = x_ref.shape[-1]
    x = x_ref[0]
    mu = jnp.mean(x, axis=-1, keepdims=True)
    xc = x - mu
    var = jnp.mean(xc * xc, axis=-1, keepdims=True)
    xn = xc * jax.lax.rsqrt(var + _EPS)
    scale = mod_ref[0, :, :d]
    shift = mod_ref[0, :, d:]
    y = xn * (1.0 + scale) + shift
    y = y * jax.nn.sigmoid(y)
    out_ref[0] = (
        jnp.dot(y, w_proj_ref[...], preferred_element_type=jnp.float32)
        + b_proj_ref[...]
    )


def kernel(x, cond, w_ada, b_ada, w_proj, b_proj):
    B, T, D = x.shape
    OUT = w_proj.shape[1]
    TBLK = 1024

    mod = pl.pallas_call(
        _mod_kernel,
        out_shape=jax.ShapeDtypeStruct((B, 2 * D), jnp.float32),
    )(cond, w_ada, b_ada.reshape(1, 2 * D))
    mod3 = mod.reshape(B, 1, 2 * D)

    mesh = pltpu.create_tensorcore_mesh("core", num_cores=2)

    def inner(refs):
        x_ref, mod_ref, w_proj_ref, b_proj_ref, out_ref = refs

        @pl.core_map(mesh)
        def _():
            pipeline = pltpu.emit_pipeline(
                _main_body,
                grid=(B, T // TBLK),
                in_specs=[
                    pl.BlockSpec((1, TBLK, D), lambda b, t: (b, t, 0)),
                    pl.BlockSpec((1, 1, 2 * D), lambda b, t: (b, 0, 0)),
                    pl.BlockSpec((D, OUT), lambda b, t: (0, 0)),
                    pl.BlockSpec((1, OUT), lambda b, t: (0, 0)),
                ],
                out_specs=[pl.BlockSpec((1, TBLK, OUT), lambda b, t: (b, t, 0))],
                core_axis_name="core",
                dimension_semantics=(pltpu.PARALLEL, pltpu.ARBITRARY),
            )
            pipeline(x_ref, mod_ref, w_proj_ref, b_proj_ref, out_ref)

    _, _, _, _, out = pl.run_state(inner)(
        (x, mod3, w_proj, b_proj.reshape(1, OUT),
         jnp.zeros((B, T, OUT), jnp.float32))
    )
    return out


# speedup vs baseline: 1.1413x; 1.1413x over previous
import jax
import jax.numpy as jnp
from jax.experimental import pallas as pl
from jax.experimental.pallas import tpu as pltpu

_EPS = 1e-6


def _mod_kernel(cond_ref, w_ada_ref, b_ada_ref, mod_ref):
    c = cond_ref[...]
    s = c * jax.nn.sigmoid(c)
    mod_ref[...] = (
        jnp.dot(s, w_ada_ref[...], preferred_element_type=jnp.float32)
        + b_ada_ref[...]
    )


def _half(x, a_b, b_b, w_proj_ref, b_proj_ref, out_ref):
    d = x.shape[-1]
    inv_d = 1.0 / d
    xb = x.astype(jnp.bfloat16)
    s1 = jnp.sum(xb, axis=-1, keepdims=True, dtype=jnp.bfloat16)
    s2 = jnp.sum(xb * xb, axis=-1, keepdims=True, dtype=jnp.bfloat16)
    mu = s1.astype(jnp.float32) * inv_d
    var = s2.astype(jnp.float32) * inv_d - mu * mu
    r = jax.lax.rsqrt(var + _EPS)
    xn = (x - mu) * r
    z = xn.astype(jnp.bfloat16) * a_b + b_b
    y = z / (1.0 + jnp.exp(-z))
    out_ref[0] = (
        jnp.dot(y, w_proj_ref[...], preferred_element_type=jnp.float32)
        + b_proj_ref[...]
    )


def _main_body(x_lo_ref, x_hi_ref, mod_ref, w_proj_ref, b_proj_ref,
               out_lo_ref, out_hi_ref):
    d = x_lo_ref.shape[-1]
    a_b = (1.0 + mod_ref[0, :, :d]).astype(jnp.bfloat16)
    b_b = mod_ref[0, :, d:].astype(jnp.bfloat16)
    _half(x_lo_ref[0], a_b, b_b, w_proj_ref, b_proj_ref, out_lo_ref)
    _half(x_hi_ref[0], a_b, b_b, w_proj_ref, b_proj_ref, out_hi_ref)


def kernel(x, cond, w_ada, b_ada, w_proj, b_proj):
    B, T, D = x.shape
    OUT = w_proj.shape[1]
    TBLK = 1024
    H = TBLK // 2

    mod = pl.pallas_call(
        _mod_kernel,
        out_shape=jax.ShapeDtypeStruct((B, 2 * D), jnp.float32),
    )(cond, w_ada, b_ada.reshape(1, 2 * D))
    mod3 = mod.reshape(B, 1, 2 * D)

    grid = (B, T // TBLK)
    out_lo, out_hi = pl.pallas_call(
        _main_body,
        out_shape=(
            jax.ShapeDtypeStruct((B, T // 2, OUT), jnp.float32),
            jax.ShapeDtypeStruct((B, T // 2, OUT), jnp.float32),
        ),
        grid=grid,
        in_specs=[
            pl.BlockSpec((1, H, D), lambda b, t: (b, 2 * t, 0)),
            pl.BlockSpec((1, H, D), lambda b, t: (b, 2 * t + 1, 0)),
            pl.BlockSpec((1, 1, 2 * D), lambda b, t: (b, 0, 0)),
            pl.BlockSpec((D, OUT), lambda b, t: (0, 0)),
            pl.BlockSpec((1, OUT), lambda b, t: (0, 0)),
        ],
        out_specs=(
            pl.BlockSpec((1, H, OUT), lambda b, t: (b, t, 0)),
            pl.BlockSpec((1, H, OUT), lambda b, t: (b, t, 0)),
        ),
        compiler_params=pltpu.CompilerParams(
            dimension_semantics=("parallel", "arbitrary"),
            vmem_limit_bytes=48 * 1024 * 1024,
        ),
    )(x, x, mod3, w_proj.astype(jnp.bfloat16), b_proj.reshape(1, OUT))
    out = jnp.stack(
        [out_lo.reshape(B, T // TBLK, H, OUT),
         out_hi.reshape(B, T // TBLK, H, OUT)], axis=2,
    ).reshape(B, T, OUT)
    return out
